# Initial kernel scaffold; baseline (speedup 1.0000x reference)
import jax
import jax.numpy as jnp
from jax.experimental import pallas as pl


def kernel(memories, importance, W1, b1, W2, b2):
    raise NotImplementedError("write your pallas kernel here")


# speedup vs baseline: 1.2410x; 1.2410x over previous
import jax
import jax.numpy as jnp
from jax.experimental import pallas as pl
from jax.experimental.pallas import tpu as pltpu


def _body(k_ref, mem_ref, w1_ref, b1_ref, w2_ref, b2_ref, out_ref):
    T = mem_ref.shape[1]
    mem = mem_ref[0]

    h = jnp.dot(mem, w1_ref[...], preferred_element_type=jnp.float32) + b1_ref[...]
    h = jnp.where(h >= 0.0, h, 0.01 * h)
    p = jnp.tanh(
        jnp.dot(h, w2_ref[...], preferred_element_type=jnp.float32) + b2_ref[...]
    )

    ones_t = jnp.ones((1, T), jnp.float32)
    c = jnp.dot(ones_t, p, preferred_element_type=jnp.float32) * (1.0 / T)

    q2 = (p - c) * (p - c)
    ones_d = jnp.ones((1, q2.shape[1]), jnp.float32)
    d2 = jax.lax.dot_general(
        ones_d, q2, (((1,), (1,)), ((), ())), preferred_element_type=jnp.float32
    )
    dist = jnp.sqrt(d2)
    bits = jax.lax.bitcast_convert_type(dist, jnp.int32)

    k = k_ref[pl.program_id(0), 0]

    def step(i, res):
        cand = res | (jnp.int32(1) << (jnp.int32(30) - i))
        cnt = jnp.sum((bits < cand).astype(jnp.int32))
        return jnp.where(cnt >= k, res, cand)

    res = jax.lax.fori_loop(0, 31, step, jnp.int32(0))

    maskf = (bits <= res).astype(jnp.float32)
    cnt = jnp.maximum(jnp.sum(maskf), 1.0)
    r = jnp.dot(maskf, p, preferred_element_type=jnp.float32)
    out_ref[0] = r / cnt


def kernel(memories, importance, W1, b1, W2, b2):
    B, T, D = memories.shape
    TOPO = W2.shape[1]
    k = jnp.clip(
        jnp.round(importance * (T - 1) + 1.0).astype(jnp.int32), 1, T
    )
    return pl.pallas_call(
        _body,
        grid=(B,),
        in_specs=[
            pl.BlockSpec((B, 1), lambda b: (0, 0), memory_space=pltpu.SMEM),
            pl.BlockSpec((1, T, D), lambda b: (b, 0, 0)),
            pl.BlockSpec((D, TOPO), lambda b: (0, 0)),
            pl.BlockSpec((1, TOPO), lambda b: (0, 0)),
            pl.BlockSpec((TOPO, TOPO), lambda b: (0, 0)),
            pl.BlockSpec((1, TOPO), lambda b: (0, 0)),
        ],
        out_specs=pl.BlockSpec((1, 1, TOPO), lambda b: (b, 0, 0)),
        out_shape=jax.ShapeDtypeStruct((B, 1, TOPO), jnp.float32),
    )(k, memories, W1, b1.reshape(1, TOPO), W2, b2.reshape(1, TOPO)).reshape(
        B, TOPO
    )


# speedup vs baseline: 3.6797x; 2.9651x over previous
import jax
import jax.numpy as jnp
from jax import lax
from jax.experimental import pallas as pl
from jax.experimental.pallas import tpu as pltpu


def _body(k_ref, mem_ref, w1_ref, b1_ref, w2_ref, b2_ref, out_ref, p_sc, bits_sc):
    B, T, _ = p_sc.shape
    pid = pl.program_id(0)

    @pl.when(pid < B)
    def _mlp():
        mem = mem_ref[0]
        h = jnp.dot(mem, w1_ref[...], preferred_element_type=jnp.float32)
        h += b1_ref[...]
        h = jnp.where(h >= 0.0, h, 0.01 * h)
        p = jnp.tanh(
            jnp.dot(h, w2_ref[...], preferred_element_type=jnp.float32) + b2_ref[...]
        )

        ones_t = jnp.ones((1, T), jnp.float32)
        c = jnp.dot(ones_t, p, preferred_element_type=jnp.float32) * (1.0 / T)
        q2 = (p - c) * (p - c)
        ones_d = jnp.ones((1, q2.shape[1]), jnp.float32)
        d2 = lax.dot_general(
            ones_d, q2, (((1,), (1,)), ((), ())), preferred_element_type=jnp.float32
        )
        dist = jnp.sqrt(d2)
        p_sc[pid] = p
        bits_sc[pid] = lax.bitcast_convert_type(dist, jnp.int32)

    @pl.when(pid == B)
    def _select():
        bits = jnp.concatenate([bits_sc[b] for b in range(B)], axis=0)
        kv = k_ref[...]

        def step(i, res):
            cand = res | (jnp.int32(1) << (jnp.int32(30) - i))
            cnt = jnp.sum((bits < cand).astype(jnp.int32), axis=1, keepdims=True)
            return jnp.where(cnt >= kv, res, cand)

        res = lax.fori_loop(0, 31, step, jnp.zeros((B, 1), jnp.int32))

        maskf = (bits <= res).astype(jnp.float32)
        counts = jnp.maximum(jnp.sum(maskf, axis=1, keepdims=True), 1.0)
        for b in range(B):
            mrow = lax.slice(maskf, (b, 0), (b + 1, T))
            cb = lax.slice(counts, (b, 0), (b + 1, 1))
            rb = jnp.dot(mrow, p_sc[b], preferred_element_type=jnp.float32)
            out_ref[pl.ds(b, 1), :] = rb / cb


def kernel(memories, importance, W1, b1, W2, b2):
    B, T, D = memories.shape
    TOPO = W2.shape[1]
    k = jnp.clip(
        jnp.round(importance * (T - 1) + 1.0).astype(jnp.int32), 1, T
    )
    return pl.pallas_call(
        _body,
        grid=(B + 1,),
        in_specs=[
            pl.BlockSpec((B, 1), lambda i: (0, 0)),
            pl.BlockSpec((1, T, D), lambda i: (jnp.minimum(i, B - 1), 0, 0)),
            pl.BlockSpec((D, TOPO), lambda i: (0, 0)),
            pl.BlockSpec((1, TOPO), lambda i: (0, 0)),
            pl.BlockSpec((TOPO, TOPO), lambda i: (0, 0)),
            pl.BlockSpec((1, TOPO), lambda i: (0, 0)),
        ],
        out_specs=pl.BlockSpec((B, TOPO), lambda i: (0, 0)),
        out_shape=jax.ShapeDtypeStruct((B, TOPO), jnp.float32),
        scratch_shapes=[
            pltpu.VMEM((B, T, TOPO), jnp.float32),
            pltpu.VMEM((B, 1, T), jnp.int32),
        ],
        compiler_params=pltpu.CompilerParams(
            vmem_limit_bytes=100 * 1024 * 1024,
        ),
    )(k, memories, W1, b1.reshape(1, TOPO), W2, b2.reshape(1, TOPO))
